# baseline (speedup 1.0000x reference)
import math

import jax
import jax.numpy as jnp
import numpy as np
from jax.experimental import pallas as pl
from jax.experimental.pallas import tpu as pltpu

EPS = 1e-06
S = 0.025
ALPHA = 0.98
DELTA = 2.0

CHUNK = 256
BLOCK_T = 8192
LANES = 128


def _pcen(xb, m):
    w = jax.lax.exp2(jnp.log2(m + EPS) * np.float32(-ALPHA))
    y = xb * w + DELTA
    return y * jax.lax.rsqrt(y) - np.float32(math.sqrt(DELTA))


def _pcen_kernel(x_ref, o_ref, l_scr, d_scr, p_scr, m_scr):
    k = pl.program_id(1)
    first = k == 0
    C = CHUNK

    @pl.when(first & (pl.program_id(0) == 0))
    def _init():
        ln_a = np.float32(math.log(1.0 - S))
        ii = jax.lax.broadcasted_iota(jnp.int32, (C, C), 0)
        jj = jax.lax.broadcasted_iota(jnp.int32, (C, C), 1)
        di = (ii - jj).astype(jnp.float32)
        l_scr[...] = jnp.where(di >= 0.0, S * jnp.exp(di * ln_a), 0.0).astype(
            jnp.bfloat16
        )
        ir = jax.lax.broadcasted_iota(jnp.int32, (C, LANES), 0).astype(jnp.float32)
        d_scr[...] = (1.0 - S) * jnp.exp(ir * ln_a)
        p_scr[...] = jnp.exp((ir + 1.0) * ln_a)

    l_mat = l_scr[...]
    p_vec = p_scr[...]

    xb = x_ref[0, 0:C, :]
    vec = jnp.where(first, d_scr[...], p_vec)
    m_prev = jnp.where(first, xb[0:1, :], m_scr[...])
    m = jax.lax.dot_general(
        l_mat, xb.astype(jnp.bfloat16), (((1,), (0,)), ((), ())),
        preferred_element_type=jnp.float32,
    ) + vec * m_prev
    o_ref[0, 0:C, :] = _pcen(xb, m)
    m_prev = m[C - 1:C, :]

    for c in range(1, BLOCK_T // C):
        xb = x_ref[0, c * C:(c + 1) * C, :]
        m = jax.lax.dot_general(
            l_mat, xb.astype(jnp.bfloat16), (((1,), (0,)), ((), ())),
            preferred_element_type=jnp.float32,
            ) + p_vec * m_prev
        o_ref[0, c * C:(c + 1) * C, :] = _pcen(xb, m)
        m_prev = m[C - 1:C, :]

    m_scr[...] = m_prev


def _pcen_call(x):
    B, T, F = x.shape
    return pl.pallas_call(
        _pcen_kernel,
        grid=(B, T // BLOCK_T),
        in_specs=[pl.BlockSpec((1, BLOCK_T, F), lambda b, t: (b, t, 0))],
        out_specs=pl.BlockSpec((1, BLOCK_T, F), lambda b, t: (b, t, 0)),
        out_shape=jax.ShapeDtypeStruct((B, T, F), jnp.float32),
        scratch_shapes=[
            pltpu.VMEM((CHUNK, CHUNK), jnp.bfloat16),
            pltpu.VMEM((CHUNK, F), jnp.float32),
            pltpu.VMEM((CHUNK, F), jnp.float32),
            pltpu.VMEM((1, F), jnp.float32),
        ],
        compiler_params=pltpu.CompilerParams(
            dimension_semantics=("arbitrary", "arbitrary"),
        ),
    )(x)


def kernel(x):
    return _pcen_call(x)


# speedup vs baseline: 1.0644x; 1.0644x over previous
import math

import jax
import jax.numpy as jnp
import numpy as np
from jax.experimental import pallas as pl
from jax.experimental.pallas import tpu as pltpu

EPS = 1e-06
S = 0.025
ALPHA = 0.98
DELTA = 2.0

CHUNK = 256
LANES = 128


def _pcen(xb, m):
    w = jax.lax.exp2(jnp.log(m + EPS) * np.float32(-ALPHA / math.log(2.0)))
    y = xb * w + DELTA
    return y * jax.lax.rsqrt(y) - np.float32(math.sqrt(DELTA))


def _pcen_kernel(x_ref, o_ref, l_scr, p_scr):
    C = CHUNK

    @pl.when(pl.program_id(0) == 0)
    def _init():
        ln_a = np.float32(math.log(1.0 - S))
        ii = jax.lax.broadcasted_iota(jnp.int32, (C, C), 0)
        jj = jax.lax.broadcasted_iota(jnp.int32, (C, C), 1)
        di = (ii - jj).astype(jnp.float32)
        l_scr[...] = jnp.where(di >= 0.0, S * jnp.exp(di * ln_a), 0.0)
        ir = jax.lax.broadcasted_iota(jnp.int32, (C, LANES), 0).astype(jnp.float32)
        p_scr[...] = jnp.exp((ir + 1.0) * ln_a)

    l_mat = l_scr[...]
    p_vec = p_scr[...]

    m_prev = x_ref[0, 0:1, :]
    for c in range(x_ref.shape[1] // C):
        xb = x_ref[0, c * C:(c + 1) * C, :]
        m = jax.lax.dot_general(
            l_mat, xb, (((1,), (0,)), ((), ())),
            preferred_element_type=jnp.float32,
        ) + p_vec * m_prev
        o_ref[0, c * C:(c + 1) * C, :] = _pcen(xb, m)
        m_prev = m[C - 1:C, :]


def kernel(x):
    B, T, F = x.shape
    return pl.pallas_call(
        _pcen_kernel,
        grid=(B,),
        in_specs=[pl.BlockSpec((1, T, F), lambda b: (b, 0, 0))],
        out_specs=pl.BlockSpec((1, T, F), lambda b: (b, 0, 0)),
        out_shape=jax.ShapeDtypeStruct((B, T, F), jnp.float32),
        scratch_shapes=[
            pltpu.VMEM((CHUNK, CHUNK), jnp.float32),
            pltpu.VMEM((CHUNK, F), jnp.float32),
        ],
        compiler_params=pltpu.CompilerParams(
            dimension_semantics=("arbitrary",),
        ),
    )(x)


# speedup vs baseline: 1.1648x; 1.0943x over previous
import math

import jax
import jax.numpy as jnp
import numpy as np
from jax.experimental import pallas as pl
from jax.experimental.pallas import tpu as pltpu

EPS = 1e-06
S = 0.025
ALPHA = 0.98
DELTA = 2.0

CHUNK = 256
LANES = 128


def _pcen(xb, m):
    w = jax.lax.exp2(jnp.log(m + EPS) * np.float32(-ALPHA / math.log(2.0)))
    y = xb * w + DELTA
    return y * jax.lax.rsqrt(y) - np.float32(math.sqrt(DELTA))


def _pcen_kernel(x_ref, o_ref, l_scr, p_scr):
    C = CHUNK

    @pl.when(pl.program_id(0) == 0)
    def _init():
        ln_a = np.float32(math.log(1.0 - S))
        ii = jax.lax.broadcasted_iota(jnp.int32, (C, C), 0)
        jj = jax.lax.broadcasted_iota(jnp.int32, (C, C), 1)
        di = (ii - jj).astype(jnp.float32)
        l_scr[...] = jnp.where(di >= 0.0, S * jnp.exp(di * ln_a), 0.0)
        ir = jax.lax.broadcasted_iota(jnp.int32, (C, LANES), 0).astype(jnp.float32)
        p_scr[...] = jnp.exp((ir + 1.0) * ln_a)

    for b2 in range(x_ref.shape[0]):
        m_prev = x_ref[b2, 0:1, :]
        for c in range(x_ref.shape[1] // C):
            xb = x_ref[b2, c * C:(c + 1) * C, :]
            m = jax.lax.dot_general(
                l_scr[...], xb, (((1,), (0,)), ((), ())),
                preferred_element_type=jnp.float32,
            ) + p_scr[...] * m_prev
            o_ref[b2, c * C:(c + 1) * C, :] = _pcen(xb, m)
            m_prev = m[C - 1:C, :]


def kernel(x):
    B, T, F = x.shape
    return pl.pallas_call(
        _pcen_kernel,
        grid=(B // 2,),
        in_specs=[pl.BlockSpec((2, T, F), lambda b: (b, 0, 0))],
        out_specs=pl.BlockSpec((2, T, F), lambda b: (b, 0, 0)),
        out_shape=jax.ShapeDtypeStruct((B, T, F), jnp.float32),
        scratch_shapes=[
            pltpu.VMEM((CHUNK, CHUNK), jnp.float32),
            pltpu.VMEM((CHUNK, F), jnp.float32),
        ],
        compiler_params=pltpu.CompilerParams(
            dimension_semantics=("arbitrary",),
        ),
    )(x)


# speedup vs baseline: 1.1739x; 1.0078x over previous
import math

import jax
import jax.numpy as jnp
import numpy as np
from jax.experimental import pallas as pl
from jax.experimental.pallas import tpu as pltpu

EPS = 1e-06
S = 0.025
ALPHA = 0.98
DELTA = 2.0

CHUNK = 256
LANES = 128


def _pcen(xb, m):
    w = jax.lax.exp2(jnp.log(m + EPS) * np.float32(-ALPHA / math.log(2.0)))
    y = xb * w + DELTA
    return y * jax.lax.rsqrt(y) - np.float32(math.sqrt(DELTA))


def _pcen_kernel(x_ref, o_ref, l_scr, p_scr, m_scr):
    C = CHUNK
    t0 = pl.program_id(1) == 0

    @pl.when((pl.program_id(0) == 0) & t0)
    def _init():
        ln_a = np.float32(math.log(1.0 - S))
        ii = jax.lax.broadcasted_iota(jnp.int32, (C, C), 0)
        jj = jax.lax.broadcasted_iota(jnp.int32, (C, C), 1)
        di = (ii - jj).astype(jnp.float32)
        l_scr[...] = jnp.where(di >= 0.0, S * jnp.exp(di * ln_a), 0.0)
        ir = jax.lax.broadcasted_iota(jnp.int32, (C, LANES), 0).astype(jnp.float32)
        p_scr[...] = jnp.exp((ir + 1.0) * ln_a)

    for b2 in range(x_ref.shape[0]):
        m_prev = jnp.where(t0, x_ref[b2, 0:1, :], m_scr[b2:b2 + 1, :])
        for c in range(x_ref.shape[1] // C):
            xb = x_ref[b2, c * C:(c + 1) * C, :]
            m = jax.lax.dot_general(
                l_scr[...], xb, (((1,), (0,)), ((), ())),
                preferred_element_type=jnp.float32,
            ) + p_scr[...] * m_prev
            o_ref[b2, c * C:(c + 1) * C, :] = _pcen(xb, m)
            m_prev = m[C - 1:C, :]
        m_scr[b2:b2 + 1, :] = m_prev


def kernel(x):
    B, T, F = x.shape
    return pl.pallas_call(
        _pcen_kernel,
        grid=(B // 4, 2),
        in_specs=[pl.BlockSpec((4, T // 2, F), lambda b, t: (b, t, 0))],
        out_specs=pl.BlockSpec((4, T // 2, F), lambda b, t: (b, t, 0)),
        out_shape=jax.ShapeDtypeStruct((B, T, F), jnp.float32),
        scratch_shapes=[
            pltpu.VMEM((CHUNK, CHUNK), jnp.float32),
            pltpu.VMEM((CHUNK, F), jnp.float32),
            pltpu.VMEM((8, F), jnp.float32),
        ],
        compiler_params=pltpu.CompilerParams(
            dimension_semantics=("arbitrary", "arbitrary"),
        ),
    )(x)
